# baseline (speedup 1.0000x reference)
import functools

import jax
import jax.numpy as jnp
from jax import lax
from jax.experimental import pallas as pl
from jax.experimental.pallas import tpu as pltpu
from jax.experimental.pallas import tpu_sc as plsc

N_NODES = 10000
N_EDGES = 320000
D = 128

NC = 2
NS = 16
NW = NC * NS

B = 80
BLK_BYTES = B * D * 4

B2 = 80
EPW = N_EDGES // NW
NB2 = EPW // B2
NBP2 = 128

N_PAD = 10240
RPT = N_PAD // NS
RC = 128
NRC = RPT // RC

_mesh = plsc.VectorSubcoreMesh(core_axis_name="c", subcore_axis_name="s")


@functools.partial(
    pl.kernel,
    out_type=jax.ShapeDtypeStruct((NC, N_PAD, D), jnp.float32),
    mesh=_mesh,
    scratch_types=[
        pltpu.VMEM((NBP2, B), jnp.int32),
        pltpu.VMEM((B, D), jnp.float32),
        pltpu.VMEM((B, D), jnp.float32),
        pltpu.VMEM_SHARED((N_PAD, D), jnp.float32),
        pltpu.SemaphoreType.DMA,
        pltpu.SemaphoreType.DMA,
    ],
)
def _phase1(idx2_hbm, x_hbm, extra_hbm, part_hbm,
            idx_v, x_v0, x_v1, table, sx0, sx1):
    c = lax.axis_index("c")
    s = lax.axis_index("s")
    wid = s * NC + c
    xv = (x_v0, x_v1)
    sx = (sx0, sx1)

    for k in range(NRC):
        ro = s * RPT + k * RC
        pltpu.sync_copy(extra_hbm.at[pl.ds(ro, RC)], table.at[pl.ds(ro, RC)])
    plsc.subcore_barrier()

    pltpu.sync_copy(idx2_hbm.at[pl.ds(wid * NBP2, NBP2)], idx_v)

    def xsrc(b):
        return x_hbm.at[pl.ds(wid * EPW + b * B, B)]

    pltpu.async_copy(xsrc(0), xv[0], sx[0])

    def pair(k, carry):
        for boff, p in ((0, 0), (1, 1)):
            b = 2 * k + boff
            q = 1 - p
            pltpu.make_async_copy(xsrc(b), xv[p], sx[p]).wait()
            pltpu.async_copy(xv[p], table.at[idx_v.at[b]], sx[p], add=True)

            @pl.when(b + 1 < NB2)
            def _():
                @pl.when(b > 0)
                def _():
                    pltpu.make_async_copy(xsrc(b), xv[q], sx[q]).wait()
                pltpu.async_copy(xsrc(b + 1), xv[q], sx[q])
        return carry

    lax.fori_loop(0, NB2 // 2, pair, 0)
    pltpu.make_async_copy(xsrc(0), xv[0], sx[0]).wait()
    pltpu.sync_copy(xv[0], table.at[idx_v.at[NB2 - 1]], add=True)
    pltpu.make_async_copy(xsrc(0), xv[1], sx[1]).wait()
    plsc.subcore_barrier()

    for k in range(NRC):
        ro = s * RPT + k * RC
        pltpu.sync_copy(table.at[pl.ds(ro, RC)],
                        part_hbm.at[c].at[pl.ds(ro, RC)])


def _merge_pack_tc(part, extra_p):
    R = 1024

    def body(p0_ref, p1_ref, ex_ref, o_ref):
        a = p0_ref[0] + p1_ref[0] - ex_ref[...]
        u = lax.bitcast_convert_type(a, jnp.int32)
        u = u + 0x7FFF + ((u >> 16) & 1)
        lo = jnp.concatenate(
            [u[:, k * 32:k * 32 + 16] for k in range(4)], axis=1)
        hi = jnp.concatenate(
            [u[:, k * 32 + 16:k * 32 + 32] for k in range(4)], axis=1)
        o_ref[...] = (hi & (-65536)) | ((lo >> 16) & 0xFFFF)

    return pl.pallas_call(
        body,
        grid=(N_PAD // R,),
        in_specs=[
            pl.BlockSpec((1, R, D), lambda i: (0, i, 0)),
            pl.BlockSpec((1, R, D), lambda i: (1, i, 0)),
            pl.BlockSpec((R, D), lambda i: (i, 0)),
        ],
        out_specs=pl.BlockSpec((R, D // 2), lambda i: (i, 0)),
        out_shape=jax.ShapeDtypeStruct((N_PAD, D // 2), jnp.int32),
    )(part, part, extra_p)


@functools.partial(
    pl.kernel,
    out_type=jax.ShapeDtypeStruct((N_EDGES, D), jnp.float32),
    mesh=_mesh,
    compiler_params=pltpu.CompilerParams(use_tc_tiling_on_sc=False),
    scratch_types=[
        pltpu.VMEM((NBP2, B2), jnp.int32),
        pltpu.VMEM((B2, D), jnp.float32),
        pltpu.VMEM((B2, D), jnp.float32),
        pltpu.VMEM((B2, D // 2), jnp.int32),
        pltpu.VMEM((B2, D // 2), jnp.int32),
        pltpu.SemaphoreType.DMA,
        pltpu.SemaphoreType.DMA,
        pltpu.SemaphoreType.DMA,
        pltpu.SemaphoreType.DMA,
    ],
)
def _phase2(idx2_hbm, x_hbm, merged_hbm, out_hbm,
            idx_v, x_v0, x_v1, g_v0, g_v1,
            sg0, sg1, so0, so1):
    c = lax.axis_index("c")
    s = lax.axis_index("s")
    wid = s * NC + c
    xv = (x_v0, x_v1)
    gv = (g_v0, g_v1)
    sg = (sg0, sg1)
    so = (so0, so1)

    pltpu.sync_copy(idx2_hbm.at[pl.ds(wid * NBP2, NBP2)], idx_v)

    def xsrc(b):
        return x_hbm.at[pl.ds(wid * EPW + b * B2, B2)]

    def odst(b):
        return out_hbm.at[pl.ds(wid * EPW + b * B2, B2)]

    def load_block(b, p):
        pltpu.async_copy(xsrc(b), xv[p], sg[p])
        pltpu.async_copy(merged_hbm.at[idx_v.at[b]], gv[p], sg[p])

    def sub(g, x):
        def row(r, carry):
            for cc in range(D // 32):
                w = g[r, pl.ds(cc * 16, 16)]
                sa = pl.ds(cc * 32, 16)
                sb = pl.ds(cc * 32 + 16, 16)
                x[r, sa] = lax.bitcast_convert_type(
                    w << 16, jnp.float32) - x[r, sa]
                x[r, sb] = lax.bitcast_convert_type(
                    w & (-65536), jnp.float32) - x[r, sb]
            return carry
        lax.fori_loop(0, B2, row, 0)

    def wait_block(b, p):
        pltpu.make_async_copy(xsrc(b), xv[p], sg[p]).wait()
        pltpu.make_async_copy(merged_hbm.at[pl.ds(0, B2)], gv[p], sg[p]).wait()

    load_block(0, 0)
    wait_block(0, 0)
    load_block(1, 1)
    sub(gv[0], xv[0])
    pltpu.async_copy(xv[0], odst(0), so[0])

    def pair(k, carry):
        for boff, p in ((1, 1), (2, 0)):
            b = 2 * k + boff
            q = 1 - p
            wait_block(b, p)

            @pl.when(b + 1 < NB2)
            def _():
                pltpu.make_async_copy(xv[q], odst(b - 1), so[q]).wait()
                load_block(b + 1, q)

            sub(gv[p], xv[p])
            pltpu.async_copy(xv[p], odst(b), so[p])
        return carry

    lax.fori_loop(0, (NB2 - 1) // 2, pair, 0)
    pltpu.make_async_copy(xv[1], odst(NB2 - 2), so[1]).wait()
    pltpu.make_async_copy(xv[0], odst(NB2 - 1), so[0]).wait()


@jax.jit
def kernel(edge_index, extra, x):
    idx = edge_index[0]
    idx_p2 = jnp.pad(idx.reshape(NW, NB2, B2), ((0, 0), (0, NBP2 - NB2), (0, 0)))
    idx_p2 = idx_p2.reshape(NW * NBP2, B2)
    extra_p = jnp.pad(extra, ((0, N_PAD - N_NODES), (0, 0)))
    part = _phase1(idx_p2, x, extra_p)
    merged = _merge_pack_tc(part, extra_p)
    return _phase2(idx_p2, x, merged)


# speedup vs baseline: 1.1417x; 1.1417x over previous
import functools

import jax
import jax.numpy as jnp
from jax import lax
from jax.experimental import pallas as pl
from jax.experimental.pallas import tpu as pltpu
from jax.experimental.pallas import tpu_sc as plsc

N_NODES = 10000
N_EDGES = 320000
D = 128

NC = 2
NS = 16
NW = NC * NS

B = 80
BLK_BYTES = B * D * 4

B2 = 80
EPW = N_EDGES // NW
NB2 = EPW // B2
NBP2 = 128

N_PAD = 10240
RPT = N_PAD // NS
RC = 128
NRC = RPT // RC

_mesh = plsc.VectorSubcoreMesh(core_axis_name="c", subcore_axis_name="s")


@functools.partial(
    pl.kernel,
    out_type=jax.ShapeDtypeStruct((NC, N_PAD, D), jnp.float32),
    mesh=_mesh,
    scratch_types=[
        pltpu.VMEM((NBP2, B), jnp.int32),
        pltpu.VMEM((B, D), jnp.float32),
        pltpu.VMEM((B, D), jnp.float32),
        pltpu.VMEM_SHARED((N_PAD, D), jnp.float32),
        pltpu.SemaphoreType.DMA,
        pltpu.SemaphoreType.DMA,
    ],
)
def _phase1(idx2_hbm, x_hbm, extra_hbm, part_hbm,
            idx_v, x_v0, x_v1, table, sx0, sx1):
    c = lax.axis_index("c")
    s = lax.axis_index("s")
    wid = s * NC + c
    xv = (x_v0, x_v1)
    sx = (sx0, sx1)

    ro = s * RPT
    pltpu.sync_copy(extra_hbm.at[pl.ds(ro, RPT)], table.at[pl.ds(ro, RPT)])
    plsc.subcore_barrier()

    pltpu.sync_copy(idx2_hbm.at[pl.ds(wid * NBP2, NBP2)], idx_v)

    def xsrc(b):
        return x_hbm.at[pl.ds(wid * EPW + b * B, B)]

    pltpu.async_copy(xsrc(0), xv[0], sx[0])

    def pair(k, carry):
        for boff, p in ((0, 0), (1, 1)):
            b = 2 * k + boff
            q = 1 - p
            pltpu.make_async_copy(xsrc(b), xv[p], sx[p]).wait()
            pltpu.async_copy(xv[p], table.at[idx_v.at[b]], sx[p], add=True)

            @pl.when(b + 1 < NB2)
            def _():
                @pl.when(b > 0)
                def _():
                    pltpu.make_async_copy(xsrc(b), xv[q], sx[q]).wait()
                pltpu.async_copy(xsrc(b + 1), xv[q], sx[q])
        return carry

    lax.fori_loop(0, NB2 // 2, pair, 0)
    pltpu.make_async_copy(xsrc(0), xv[0], sx[0]).wait()
    pltpu.sync_copy(xv[0], table.at[idx_v.at[NB2 - 1]], add=True)
    pltpu.make_async_copy(xsrc(0), xv[1], sx[1]).wait()
    plsc.subcore_barrier()

    pltpu.sync_copy(table.at[pl.ds(ro, RPT)],
                    part_hbm.at[c].at[pl.ds(ro, RPT)])


MR = N_PAD // NW // 2


@functools.partial(
    pl.kernel,
    out_type=jax.ShapeDtypeStruct((N_PAD, D), jnp.float32),
    mesh=_mesh,
    scratch_types=[
        pltpu.VMEM((MR, D), jnp.float32),
        pltpu.VMEM((MR, D), jnp.float32),
        pltpu.VMEM((MR, D), jnp.float32),
    ],
)
def _merge(part_hbm, extra_hbm, merged_hbm, b0, b1, be):
    c = lax.axis_index("c")
    s = lax.axis_index("s")
    wid = s * NC + c

    for k in range(2):
        ro = wid * (2 * MR) + k * MR
        pltpu.sync_copy(part_hbm.at[0].at[pl.ds(ro, MR)], b0)
        pltpu.sync_copy(part_hbm.at[1].at[pl.ds(ro, MR)], b1)
        pltpu.sync_copy(extra_hbm.at[pl.ds(ro, MR)], be)

        def row(r, carry):
            for cc in range(D // 16):
                sl = pl.ds(cc * 16, 16)
                b0[r, sl] = b0[r, sl] + b1[r, sl] - be[r, sl]
            return carry

        lax.fori_loop(0, MR, row, 0)
        pltpu.sync_copy(b0, merged_hbm.at[pl.ds(ro, MR)])


@functools.partial(
    pl.kernel,
    out_type=jax.ShapeDtypeStruct((N_EDGES, D), jnp.float32),
    mesh=_mesh,
    scratch_types=[
        pltpu.VMEM((NBP2, B2), jnp.int32),
        pltpu.VMEM((B2, D), jnp.float32),
        pltpu.VMEM((B2, D), jnp.float32),
        pltpu.VMEM((B2, D), jnp.float32),
        pltpu.VMEM((B2, D), jnp.float32),
        pltpu.SemaphoreType.DMA,
        pltpu.SemaphoreType.DMA,
        pltpu.SemaphoreType.DMA,
        pltpu.SemaphoreType.DMA,
    ],
)
def _phase2(idx2_hbm, x_hbm, merged_hbm, out_hbm,
            idx_v, x_v0, x_v1, g_v0, g_v1,
            sg0, sg1, so0, so1):
    c = lax.axis_index("c")
    s = lax.axis_index("s")
    wid = s * NC + c
    xv = (x_v0, x_v1)
    gv = (g_v0, g_v1)
    sg = (sg0, sg1)
    so = (so0, so1)

    pltpu.sync_copy(idx2_hbm.at[pl.ds(wid * NBP2, NBP2)], idx_v)

    def xsrc(b):
        return x_hbm.at[pl.ds(wid * EPW + b * B2, B2)]

    def odst(b):
        return out_hbm.at[pl.ds(wid * EPW + b * B2, B2)]

    def load_block(b, p):
        pltpu.async_copy(xsrc(b), xv[p], sg[p])
        pltpu.async_copy(merged_hbm.at[idx_v.at[b]], gv[p], sg[p])

    def sub(g, x):
        def row(r, carry):
            for cc in range(D // 16):
                sl = pl.ds(cc * 16, 16)
                g[r, sl] = g[r, sl] - x[r, sl]
            return carry
        lax.fori_loop(0, B2, row, 0)

    def wait_block(b, p):
        pltpu.make_async_copy(xsrc(b), xv[p], sg[p]).wait()
        pltpu.make_async_copy(xsrc(b), gv[p], sg[p]).wait()

    load_block(0, 0)
    wait_block(0, 0)
    load_block(1, 1)
    sub(gv[0], xv[0])
    pltpu.async_copy(gv[0], odst(0), so[0])

    def pair(k, carry):
        for boff, p in ((1, 1), (2, 0)):
            b = 2 * k + boff
            q = 1 - p
            wait_block(b, p)

            @pl.when(b + 1 < NB2)
            def _():
                pltpu.make_async_copy(gv[q], odst(b - 1), so[q]).wait()
                load_block(b + 1, q)

            sub(gv[p], xv[p])
            pltpu.async_copy(gv[p], odst(b), so[p])
        return carry

    lax.fori_loop(0, (NB2 - 1) // 2, pair, 0)
    pltpu.make_async_copy(gv[1], odst(NB2 - 2), so[1]).wait()
    pltpu.make_async_copy(gv[0], odst(NB2 - 1), so[0]).wait()


@jax.jit
def kernel(edge_index, extra, x):
    idx = edge_index[0]
    idx_p2 = jnp.pad(idx.reshape(NW, NB2, B2), ((0, 0), (0, NBP2 - NB2), (0, 0)))
    idx_p2 = idx_p2.reshape(NW * NBP2, B2)
    extra_p = jnp.pad(extra, ((0, N_PAD - N_NODES), (0, 0)))
    part = _phase1(idx_p2, x, extra_p)
    merged = _merge(part, extra_p)
    return _phase2(idx_p2, x, merged)
